# baseline (speedup 1.0000x reference)
import functools

import jax
import jax.numpy as jnp
from jax import lax
from jax.experimental import pallas as pl
from jax.experimental.pallas import tpu as pltpu
from jax.experimental.pallas import tpu_sc as plsc

EPS = 1e-8

NC = 2
NS = 16
LANES = 16

CHUNK = 80
NBUF = 4
ROW_BLOCK = 1000


def _matmul_body(x_ref, w_ref, b_ref, out_ref):
    x = x_ref[...]
    w = w_ref[...]
    p = lax.dot_general(x, w, (((1,), (1,)), ((), ())),
                        preferred_element_type=jnp.float32)
    p = p + b_ref[0:1, :]
    t = out_ref.shape[0]
    d = out_ref.shape[2]
    for i in range(t):
        out_ref[i] = p[:, i * d:(i + 1) * d]


def _transform(node_states, W, b):
    n, d = node_states.shape
    t = W.shape[0] // d
    bb = jnp.broadcast_to(b.reshape(1, -1), (8, t * d))
    grid = n // ROW_BLOCK
    return pl.pallas_call(
        _matmul_body,
        grid=(grid,),
        in_specs=[
            pl.BlockSpec((ROW_BLOCK, d), lambda i: (i, 0)),
            pl.BlockSpec((t * d, d), lambda i: (0, 0)),
            pl.BlockSpec((8, t * d), lambda i: (0, 0)),
        ],
        out_specs=pl.BlockSpec((t, ROW_BLOCK, d), lambda i: (0, i, 0)),
        out_shape=jax.ShapeDtypeStruct((t, n, d), jnp.float32),
    )(node_states, W, bb)


def _sc_body(n_nodes, n_spad, dh, steps, tiles_per_type,
             table, src, tgt, out,
             src_buf, tgt_buf, rows, ones_v,
             acc_sh, cnt_sh, gsems, ssems):
    c = lax.axis_index("c")
    s = lax.axis_index("s")

    pltpu.sync_copy(src.at[s], src_buf)
    pltpu.sync_copy(tgt.at[s], tgt_buf)

    zero16 = jnp.zeros((LANES,), jnp.float32)
    type_off = jnp.where(s >= tiles_per_type, 2 * n_nodes, 0)
    off = (type_off + c).astype(jnp.int32)
    offv = jnp.full((LANES,), 1, jnp.int32) * off

    def add_off(g, _):
        def inner(l, _):
            sl = pl.ds(l * LANES, LANES)
            v = src_buf[g, sl]
            src_buf[g, sl] = v + v + offv
            return 0
        return lax.fori_loop(0, CHUNK // LANES, inner, 0)
    lax.fori_loop(0, steps, add_off, 0)

    def zrow(i, _):
        def zl(l, _):
            rows[0][i, pl.ds(l * LANES, LANES)] = zero16
            return 0
        return lax.fori_loop(0, dh // LANES, zl, 0)
    lax.fori_loop(0, CHUNK, zrow, 0)

    def zcnt(i, _):
        ones_v[i] = zero16
        return 0
    lax.fori_loop(0, CHUNK, zcnt, 0)

    zrows_per_tile = n_spad // NS
    zbase = s * zrows_per_tile
    full, rem = divmod(zrows_per_tile, CHUNK)
    for r in range(full):
        pltpu.sync_copy(rows[0], acc_sh.at[pl.ds(zbase + r * CHUNK, CHUNK)])
        pltpu.sync_copy(ones_v, cnt_sh.at[pl.ds(zbase + r * CHUNK, CHUNK)])
    if rem:
        pltpu.sync_copy(rows[0].at[pl.ds(0, rem)],
                        acc_sh.at[pl.ds(zbase + full * CHUNK, rem)])
        pltpu.sync_copy(ones_v.at[pl.ds(0, rem)],
                        cnt_sh.at[pl.ds(zbase + full * CHUNK, rem)])

    one16 = jnp.ones((LANES,), jnp.float32)

    def frow(i, _):
        ones_v[i] = one16
        return 0
    lax.fori_loop(0, CHUNK, frow, 0)

    plsc.subcore_barrier()

    def gather(j, b):
        pltpu.async_copy(table.at[src_buf.at[j]], rows[b], gsems[b])

    def gather_wait(j, b):
        pltpu.make_async_copy(table.at[src_buf.at[j]], rows[b],
                              gsems[b]).wait()

    def scatter(j, b, parity):
        del parity
        pltpu.async_copy(rows[b], acc_sh.at[tgt_buf.at[j]], ssems[b],
                         add=True)
        pltpu.async_copy(ones_v, cnt_sh.at[tgt_buf.at[j]], ssems[b],
                         add=True)

    def scatter_wait(j, b, parity):
        del parity
        pltpu.make_async_copy(rows[b], acc_sh.at[tgt_buf.at[j]],
                              ssems[b]).wait()
        pltpu.make_async_copy(ones_v, cnt_sh.at[tgt_buf.at[j]],
                              ssems[b]).wait()

    gather(0, 0)
    gather(1, 1)

    def group(v, _):
        jj = NBUF * v
        for b in range(NBUF):
            j = jj + b
            gather_wait(j, b)

            @pl.when(j >= 2)
            def _():
                scatter_wait(j - 2, (b + 2) % NBUF, b % 2)

            @pl.when(j + 2 < steps)
            def _():
                gather(j + 2, (b + 2) % NBUF)

            scatter(j, b, b % 2)
        return 0

    lax.fori_loop(0, steps // NBUF, group, 0)
    for j in range((steps // NBUF) * NBUF, steps):
        b = j % NBUF
        gather_wait(j, b)
        scatter_wait(j - 2, (b + 2) % NBUF, b % 2)
        if j + 2 < steps:
            gather(j + 2, (b + 2) % NBUF)
        scatter(j, b, b % 2)
    for j in range(steps - 2, steps):
        scatter_wait(j, j % NBUF, j % 2)

    plsc.subcore_barrier()

    rows_per_tile = n_nodes // NS
    base = s * rows_per_tile
    eps16 = jnp.full((LANES,), EPS, jnp.float32)
    fullw, remw = divmod(rows_per_tile, CHUNK)
    pieces = [(r * CHUNK, CHUNK) for r in range(fullw)]
    if remw:
        pieces.append((fullw * CHUNK, remw))
    for off, ln in pieces:
        pltpu.sync_copy(acc_sh.at[pl.ds(base + off, ln)],
                        rows[0].at[pl.ds(0, ln)])
        pltpu.sync_copy(cnt_sh.at[pl.ds(base + off, ln)],
                        ones_v.at[pl.ds(0, ln)])

        def div_row(v, _):
            cv = ones_v[v]
            dv = jnp.where(cv == 0.0, 1.0, cv)
            rv = 1.0 / dv
            for l in range(dh // LANES):
                sl = pl.ds(l * LANES, LANES)
                rows[0][v, sl] = rows[0][v, sl] * rv + eps16
            return 0
        lax.fori_loop(0, ln, div_row, 0)

        pltpu.sync_copy(rows[0].at[pl.ds(0, ln)],
                        out.at[pl.ds(base + off, ln), c])


def _sc_scatter(table, src, tgt, n_nodes, n_spad, dh, steps, tiles_per_type):
    mesh = plsc.VectorSubcoreMesh(core_axis_name="c", subcore_axis_name="s",
                                  num_cores=NC, num_subcores=NS)
    fn = pl.kernel(
        functools.partial(_sc_body, n_nodes, n_spad, dh, steps,
                          tiles_per_type),
        out_type=jax.ShapeDtypeStruct((n_nodes, NC, dh), jnp.float32),
        mesh=mesh,
        scratch_types=(
            pltpu.VMEM((steps, CHUNK), jnp.int32),
            pltpu.VMEM((steps, CHUNK), jnp.int32),
            tuple(pltpu.VMEM((CHUNK, dh), jnp.float32) for _ in range(NBUF)),
            pltpu.VMEM((CHUNK, LANES), jnp.float32),
            pltpu.VMEM_SHARED((n_spad, dh), jnp.float32),
            pltpu.VMEM_SHARED((n_spad, LANES), jnp.float32),
            tuple(pltpu.SemaphoreType.DMA for _ in range(NBUF)),
            tuple(pltpu.SemaphoreType.DMA for _ in range(NBUF)),
        ),
        compiler_params=pltpu.CompilerParams(use_tc_tiling_on_sc=False),
    )
    return fn(table, src, tgt)


def _interleave_body(h_ref, out_ref):
    dh = h_ref.shape[2]
    for h in range(NC):
        out_ref[:, h * dh:(h + 1) * dh] = h_ref[h]


def _interleave(halves, n):
    dh = halves.shape[2]
    grid = n // ROW_BLOCK
    return pl.pallas_call(
        _interleave_body,
        grid=(grid,),
        in_specs=[pl.BlockSpec((NC, ROW_BLOCK, dh), lambda i: (0, i, 0))],
        out_specs=pl.BlockSpec((ROW_BLOCK, NC * dh), lambda i: (i, 0)),
        out_shape=jax.ShapeDtypeStruct((n, NC * dh), jnp.float32),
    )(halves)


@jax.jit
def kernel(edge_lists, node_states, W, b):
    t, m, _ = edge_lists.shape
    n_nodes, dim = node_states.shape
    dh = dim // NC
    edges_per_tile = t * m // NS
    pad = (-edges_per_tile) % CHUNK
    steps = (edges_per_tile + pad) // CHUNK
    tiles_per_type = NS // t

    el = edge_lists.astype(jnp.int32)
    src = jnp.pad(el[..., 0].reshape(NS, edges_per_tile), ((0, 0), (0, pad)))
    trash = n_nodes + jnp.arange(pad, dtype=jnp.int32)
    tgt = jnp.concatenate(
        [el[..., 1].reshape(NS, edges_per_tile),
         jnp.broadcast_to(trash, (NS, pad))], axis=1)
    src = src.reshape(NS, steps, CHUNK)
    tgt = tgt.reshape(NS, steps, CHUNK)

    n_spad = n_nodes + ((pad + NS - 1) // NS) * NS

    table = _transform(node_states, W, b).reshape(NC * t * n_nodes, dh)
    out = _sc_scatter(table, src, tgt, n_nodes, n_spad, dh, steps,
                      tiles_per_type)
    return out.reshape(n_nodes, dim)


# speedup vs baseline: 1.2830x; 1.2830x over previous
import functools

import jax
import jax.numpy as jnp
from jax import lax
from jax.experimental import pallas as pl
from jax.experimental.pallas import tpu as pltpu
from jax.experimental.pallas import tpu_sc as plsc

EPS = 1e-8

NC = 2
NS = 16
LANES = 16

CHUNK = 80
NBUF = 5
ROW_BLOCK = 1000


def _matmul_body(x_ref, w_ref, b_ref, out_ref):
    x = x_ref[...]
    w = w_ref[...]
    p = lax.dot_general(x, w, (((1,), (1,)), ((), ())),
                        preferred_element_type=jnp.float32)
    p = p + b_ref[0:1, :]
    t = out_ref.shape[0]
    d = out_ref.shape[2]
    for i in range(t):
        out_ref[i] = p[:, i * d:(i + 1) * d]


def _transform(node_states, W, b):
    n, d = node_states.shape
    t = W.shape[0] // d
    bb = jnp.broadcast_to(b.reshape(1, -1), (8, t * d))
    grid = n // ROW_BLOCK
    return pl.pallas_call(
        _matmul_body,
        grid=(grid,),
        in_specs=[
            pl.BlockSpec((ROW_BLOCK, d), lambda i: (i, 0)),
            pl.BlockSpec((t * d, d), lambda i: (0, 0)),
            pl.BlockSpec((8, t * d), lambda i: (0, 0)),
        ],
        out_specs=pl.BlockSpec((t, ROW_BLOCK, d), lambda i: (0, i, 0)),
        out_shape=jax.ShapeDtypeStruct((t, n, d), jnp.float32),
    )(node_states, W, bb)


def _sc_body(n_nodes, n_spad, dh, steps, tiles_per_type,
             table, src, tgt, acc_out, cnt_out,
             src_buf, tgt_buf, rows, ones_v,
             acc_sh, cnt_sh, gsems, ssems):
    c = lax.axis_index("c")
    s = lax.axis_index("s")

    pltpu.sync_copy(src.at[s], src_buf)
    pltpu.sync_copy(tgt.at[s], tgt_buf)

    zero16 = jnp.zeros((LANES,), jnp.float32)
    type_off = jnp.where(s >= tiles_per_type, 2 * n_nodes, 0)
    off = (type_off + c).astype(jnp.int32)
    offv = jnp.full((LANES,), 1, jnp.int32) * off

    def add_off(g, _):
        def inner(l, _):
            sl = pl.ds(l * LANES, LANES)
            v = src_buf[g, sl]
            src_buf[g, sl] = v + v + offv
            return 0
        return lax.fori_loop(0, CHUNK // LANES, inner, 0)
    lax.fori_loop(0, steps, add_off, 0)

    def zrow(i, _):
        def zl(l, _):
            rows[0][i, pl.ds(l * LANES, LANES)] = zero16
            return 0
        return lax.fori_loop(0, dh // LANES, zl, 0)
    lax.fori_loop(0, CHUNK, zrow, 0)

    def zcnt(i, _):
        ones_v[i] = zero16
        return 0
    lax.fori_loop(0, CHUNK, zcnt, 0)

    zrows_per_tile = n_spad // NS
    zbase = s * zrows_per_tile
    full, rem = divmod(zrows_per_tile, CHUNK)
    for r in range(full):
        pltpu.sync_copy(rows[0], acc_sh.at[pl.ds(zbase + r * CHUNK, CHUNK)])
        pltpu.sync_copy(ones_v, cnt_sh.at[pl.ds(zbase + r * CHUNK, CHUNK)])
    if rem:
        pltpu.sync_copy(rows[0].at[pl.ds(0, rem)],
                        acc_sh.at[pl.ds(zbase + full * CHUNK, rem)])
        pltpu.sync_copy(ones_v.at[pl.ds(0, rem)],
                        cnt_sh.at[pl.ds(zbase + full * CHUNK, rem)])

    one16 = jnp.ones((LANES,), jnp.float32)

    def frow(i, _):
        ones_v[i] = one16
        return 0
    lax.fori_loop(0, CHUNK, frow, 0)

    plsc.subcore_barrier()

    def gather(j, b):
        pltpu.async_copy(table.at[src_buf.at[j]], rows[b], gsems[b])

    def gather_wait(j, b):
        pltpu.make_async_copy(table.at[src_buf.at[j]], rows[b],
                              gsems[b]).wait()

    def scatter(j, b, parity):
        pltpu.async_copy(rows[b], acc_sh.at[tgt_buf.at[j]], ssems[b],
                         add=True)

        @pl.when(c == parity)
        def _():
            pltpu.async_copy(ones_v, cnt_sh.at[tgt_buf.at[j]], ssems[b],
                             add=True)

    def scatter_wait(j, b, parity):
        pltpu.make_async_copy(rows[b], acc_sh.at[tgt_buf.at[j]],
                              ssems[b]).wait()

        @pl.when(c == parity)
        def _():
            pltpu.make_async_copy(ones_v, cnt_sh.at[tgt_buf.at[j]],
                                  ssems[b]).wait()

    depth = NBUF - 2
    for j in range(depth):
        gather(j, j)

    def group(v, _):
        jj = NBUF * v
        for b in range(NBUF):
            j = jj + b
            gather_wait(j, b)

            @pl.when(j >= 2)
            def _():
                scatter_wait(j - 2, (b + NBUF - 2) % NBUF, lax.rem(j, 2))

            @pl.when(j + depth < steps)
            def _():
                gather(j + depth, (b + depth) % NBUF)

            scatter(j, b, lax.rem(j, 2))
        return 0

    lax.fori_loop(0, steps // NBUF, group, 0)
    for j in range((steps // NBUF) * NBUF, steps):
        b = j % NBUF
        gather_wait(j, b)
        scatter_wait(j - 2, (b + NBUF - 2) % NBUF, (j - 2) % 2)
        if j + depth < steps:
            gather(j + depth, (b + depth) % NBUF)
        scatter(j, b, j % 2)
    for j in range(steps - 2, steps):
        scatter_wait(j, j % NBUF, j % 2)

    plsc.subcore_barrier()

    rows_per_tile = n_nodes // NS
    base = s * rows_per_tile
    pltpu.sync_copy(acc_sh.at[pl.ds(base, rows_per_tile)],
                    acc_out.at[c, pl.ds(base, rows_per_tile)])
    pltpu.sync_copy(cnt_sh.at[pl.ds(base, rows_per_tile)],
                    cnt_out.at[c, pl.ds(base, rows_per_tile)])


def _sc_scatter(table, src, tgt, n_nodes, n_spad, dh, steps, tiles_per_type):
    mesh = plsc.VectorSubcoreMesh(core_axis_name="c", subcore_axis_name="s",
                                  num_cores=NC, num_subcores=NS)
    fn = pl.kernel(
        functools.partial(_sc_body, n_nodes, n_spad, dh, steps,
                          tiles_per_type),
        out_type=(
            jax.ShapeDtypeStruct((NC, n_nodes, dh), jnp.float32),
            jax.ShapeDtypeStruct((NC, n_nodes, LANES), jnp.float32),
        ),
        mesh=mesh,
        scratch_types=(
            pltpu.VMEM((steps, CHUNK), jnp.int32),
            pltpu.VMEM((steps, CHUNK), jnp.int32),
            tuple(pltpu.VMEM((CHUNK, dh), jnp.float32) for _ in range(NBUF)),
            pltpu.VMEM((CHUNK, LANES), jnp.float32),
            pltpu.VMEM_SHARED((n_spad, dh), jnp.float32),
            pltpu.VMEM_SHARED((n_spad, LANES), jnp.float32),
            tuple(pltpu.SemaphoreType.DMA for _ in range(NBUF)),
            tuple(pltpu.SemaphoreType.DMA for _ in range(NBUF)),
        ),
        compiler_params=pltpu.CompilerParams(use_tc_tiling_on_sc=False),
    )
    return fn(table, src, tgt)


def _combine_body(acc_ref, cnt_ref, out_ref):
    cc = cnt_ref[0, :, 0:1] + cnt_ref[1, :, 0:1]
    div = jnp.where(cc == 0.0, 1.0, cc)
    dh = acc_ref.shape[2]
    for h in range(NC):
        out_ref[:, h * dh:(h + 1) * dh] = acc_ref[h] / div + EPS


def _combine(acc, cnt, n):
    dh = acc.shape[2]
    grid = n // ROW_BLOCK
    return pl.pallas_call(
        _combine_body,
        grid=(grid,),
        in_specs=[
            pl.BlockSpec((NC, ROW_BLOCK, dh), lambda i: (0, i, 0)),
            pl.BlockSpec((NC, ROW_BLOCK, LANES), lambda i: (0, i, 0)),
        ],
        out_specs=pl.BlockSpec((ROW_BLOCK, NC * dh), lambda i: (i, 0)),
        out_shape=jax.ShapeDtypeStruct((n, NC * dh), jnp.float32),
    )(acc, cnt)


@jax.jit
def kernel(edge_lists, node_states, W, b):
    t, m, _ = edge_lists.shape
    n_nodes, dim = node_states.shape
    dh = dim // NC
    edges_per_tile = t * m // NS
    pad = (-edges_per_tile) % CHUNK
    steps = (edges_per_tile + pad) // CHUNK
    tiles_per_type = NS // t

    el = edge_lists.astype(jnp.int32)
    src = jnp.pad(el[..., 0].reshape(NS, edges_per_tile), ((0, 0), (0, pad)))
    trash = n_nodes + jnp.arange(pad, dtype=jnp.int32)
    tgt = jnp.concatenate(
        [el[..., 1].reshape(NS, edges_per_tile),
         jnp.broadcast_to(trash, (NS, pad))], axis=1)
    src = src.reshape(NS, steps, CHUNK)
    tgt = tgt.reshape(NS, steps, CHUNK)

    n_spad = n_nodes + ((pad + NS - 1) // NS) * NS

    table = _transform(node_states, W, b).reshape(NC * t * n_nodes, dh)
    acc, cnt = _sc_scatter(table, src, tgt, n_nodes, n_spad, dh, steps,
                           tiles_per_type)
    return _combine(acc, cnt, n_nodes)


# speedup vs baseline: 1.3411x; 1.0453x over previous
import functools

import jax
import jax.numpy as jnp
from jax import lax
from jax.experimental import pallas as pl
from jax.experimental.pallas import tpu as pltpu
from jax.experimental.pallas import tpu_sc as plsc

EPS = 1e-8

NC = 2
NS = 16
LANES = 16

CHUNK = 80
NBUF = 6
ROW_BLOCK = 1000


def _matmul_body(x_ref, w_ref, b_ref, out_ref):
    x = x_ref[...]
    w = w_ref[...]
    p = lax.dot_general(x, w, (((1,), (1,)), ((), ())),
                        preferred_element_type=jnp.float32)
    p = p + b_ref[0:1, :]
    t = out_ref.shape[0]
    d = out_ref.shape[2]
    for i in range(t):
        out_ref[i] = p[:, i * d:(i + 1) * d]


def _transform(node_states, W, b):
    n, d = node_states.shape
    t = W.shape[0] // d
    bb = jnp.broadcast_to(b.reshape(1, -1), (8, t * d))
    grid = n // ROW_BLOCK
    return pl.pallas_call(
        _matmul_body,
        grid=(grid,),
        in_specs=[
            pl.BlockSpec((ROW_BLOCK, d), lambda i: (i, 0)),
            pl.BlockSpec((t * d, d), lambda i: (0, 0)),
            pl.BlockSpec((8, t * d), lambda i: (0, 0)),
        ],
        out_specs=pl.BlockSpec((t, ROW_BLOCK, d), lambda i: (0, i, 0)),
        out_shape=jax.ShapeDtypeStruct((t, n, d), jnp.float32),
    )(node_states, W, bb)


def _sc_body(n_nodes, n_spad, dh, steps, tiles_per_type,
             table, src, tgt, acc_out, cnt_out,
             src_buf, tgt_buf, rows, ones_v,
             acc_sh, cnt_sh, gsems, ssems):
    c = lax.axis_index("c")
    s = lax.axis_index("s")

    pltpu.sync_copy(src.at[s], src_buf)
    pltpu.sync_copy(tgt.at[s], tgt_buf)

    zero16 = jnp.zeros((LANES,), jnp.float32)
    type_off = jnp.where(s >= tiles_per_type, 2 * n_nodes, 0)
    off = (type_off + c).astype(jnp.int32)
    offv = jnp.full((LANES,), 1, jnp.int32) * off

    def add_off(g, _):
        def inner(l, _):
            sl = pl.ds(l * LANES, LANES)
            v = src_buf[g, sl]
            src_buf[g, sl] = v + v + offv
            return 0
        return lax.fori_loop(0, CHUNK // LANES, inner, 0)
    lax.fori_loop(0, steps, add_off, 0)

    def zrow(i, _):
        def zl(l, _):
            rows[0][i, pl.ds(l * LANES, LANES)] = zero16
            return 0
        return lax.fori_loop(0, dh // LANES, zl, 0)
    lax.fori_loop(0, CHUNK, zrow, 0)

    def zcnt(i, _):
        ones_v[i] = zero16
        return 0
    lax.fori_loop(0, CHUNK, zcnt, 0)

    zrows_per_tile = n_spad // NS
    zbase = s * zrows_per_tile
    full, rem = divmod(zrows_per_tile, CHUNK)
    for r in range(full):
        pltpu.sync_copy(rows[0], acc_sh.at[pl.ds(zbase + r * CHUNK, CHUNK)])
        pltpu.sync_copy(ones_v, cnt_sh.at[pl.ds(zbase + r * CHUNK, CHUNK)])
    if rem:
        pltpu.sync_copy(rows[0].at[pl.ds(0, rem)],
                        acc_sh.at[pl.ds(zbase + full * CHUNK, rem)])
        pltpu.sync_copy(ones_v.at[pl.ds(0, rem)],
                        cnt_sh.at[pl.ds(zbase + full * CHUNK, rem)])

    one16 = jnp.ones((LANES,), jnp.float32)

    def frow(i, _):
        ones_v[i] = one16
        return 0
    lax.fori_loop(0, CHUNK, frow, 0)

    plsc.subcore_barrier()

    def gather(j, b):
        pltpu.async_copy(table.at[src_buf.at[j]], rows[b], gsems[b])

    def gather_wait(j, b):
        pltpu.make_async_copy(table.at[src_buf.at[j]], rows[b],
                              gsems[b]).wait()

    def scatter(j, b, parity):
        pltpu.async_copy(rows[b], acc_sh.at[tgt_buf.at[j]], ssems[b],
                         add=True)

        @pl.when(c == parity)
        def _():
            pltpu.async_copy(ones_v, cnt_sh.at[tgt_buf.at[j]], ssems[b],
                             add=True)

    def scatter_wait(j, b, parity):
        pltpu.make_async_copy(rows[b], acc_sh.at[tgt_buf.at[j]],
                              ssems[b]).wait()

        @pl.when(c == parity)
        def _():
            pltpu.make_async_copy(ones_v, cnt_sh.at[tgt_buf.at[j]],
                                  ssems[b]).wait()

    depth = NBUF - 2
    for j in range(depth):
        gather(j, j)

    def group(v, _):
        jj = NBUF * v
        for b in range(NBUF):
            j = jj + b
            gather_wait(j, b)

            @pl.when(j >= 2)
            def _():
                scatter_wait(j - 2, (b + NBUF - 2) % NBUF, lax.rem(j, 2))

            @pl.when(j + depth < steps)
            def _():
                gather(j + depth, (b + depth) % NBUF)

            scatter(j, b, lax.rem(j, 2))
        return 0

    lax.fori_loop(0, steps // NBUF, group, 0)
    for j in range((steps // NBUF) * NBUF, steps):
        b = j % NBUF
        gather_wait(j, b)
        scatter_wait(j - 2, (b + NBUF - 2) % NBUF, (j - 2) % 2)
        if j + depth < steps:
            gather(j + depth, (b + depth) % NBUF)
        scatter(j, b, j % 2)
    for j in range(steps - 2, steps):
        scatter_wait(j, j % NBUF, j % 2)

    plsc.subcore_barrier()

    rows_per_tile = n_nodes // NS
    base = s * rows_per_tile
    pltpu.sync_copy(acc_sh.at[pl.ds(base, rows_per_tile)],
                    acc_out.at[c, pl.ds(base, rows_per_tile)])
    pltpu.sync_copy(cnt_sh.at[pl.ds(base, rows_per_tile)],
                    cnt_out.at[c, pl.ds(base, rows_per_tile)])


def _sc_scatter(table, src, tgt, n_nodes, n_spad, dh, steps, tiles_per_type):
    mesh = plsc.VectorSubcoreMesh(core_axis_name="c", subcore_axis_name="s",
                                  num_cores=NC, num_subcores=NS)
    fn = pl.kernel(
        functools.partial(_sc_body, n_nodes, n_spad, dh, steps,
                          tiles_per_type),
        out_type=(
            jax.ShapeDtypeStruct((NC, n_nodes, dh), jnp.float32),
            jax.ShapeDtypeStruct((NC, n_nodes, LANES), jnp.float32),
        ),
        mesh=mesh,
        scratch_types=(
            pltpu.VMEM((steps, CHUNK), jnp.int32),
            pltpu.VMEM((steps, CHUNK), jnp.int32),
            tuple(pltpu.VMEM((CHUNK, dh), jnp.float32) for _ in range(NBUF)),
            pltpu.VMEM((CHUNK, LANES), jnp.float32),
            pltpu.VMEM_SHARED((n_spad, dh), jnp.float32),
            pltpu.VMEM_SHARED((n_spad, LANES), jnp.float32),
            tuple(pltpu.SemaphoreType.DMA for _ in range(NBUF)),
            tuple(pltpu.SemaphoreType.DMA for _ in range(NBUF)),
        ),
        compiler_params=pltpu.CompilerParams(use_tc_tiling_on_sc=False),
    )
    return fn(table, src, tgt)


def _combine_body(acc_ref, cnt_ref, out_ref):
    cc = cnt_ref[0, :, 0:1] + cnt_ref[1, :, 0:1]
    div = jnp.where(cc == 0.0, 1.0, cc)
    dh = acc_ref.shape[2]
    for h in range(NC):
        out_ref[:, h * dh:(h + 1) * dh] = acc_ref[h] / div + EPS


def _combine(acc, cnt, n):
    dh = acc.shape[2]
    grid = n // ROW_BLOCK
    return pl.pallas_call(
        _combine_body,
        grid=(grid,),
        in_specs=[
            pl.BlockSpec((NC, ROW_BLOCK, dh), lambda i: (0, i, 0)),
            pl.BlockSpec((NC, ROW_BLOCK, LANES), lambda i: (0, i, 0)),
        ],
        out_specs=pl.BlockSpec((ROW_BLOCK, NC * dh), lambda i: (i, 0)),
        out_shape=jax.ShapeDtypeStruct((n, NC * dh), jnp.float32),
    )(acc, cnt)


@jax.jit
def kernel(edge_lists, node_states, W, b):
    t, m, _ = edge_lists.shape
    n_nodes, dim = node_states.shape
    dh = dim // NC
    edges_per_tile = t * m // NS
    pad = (-edges_per_tile) % CHUNK
    steps = (edges_per_tile + pad) // CHUNK
    tiles_per_type = NS // t

    el = edge_lists.astype(jnp.int32)
    src = jnp.pad(el[..., 0].reshape(NS, edges_per_tile), ((0, 0), (0, pad)))
    trash = n_nodes + jnp.arange(pad, dtype=jnp.int32)
    tgt = jnp.concatenate(
        [el[..., 1].reshape(NS, edges_per_tile),
         jnp.broadcast_to(trash, (NS, pad))], axis=1)
    src = src.reshape(NS, steps, CHUNK)
    tgt = tgt.reshape(NS, steps, CHUNK)

    n_spad = n_nodes + ((pad + NS - 1) // NS) * NS

    table = _transform(node_states, W, b).reshape(NC * t * n_nodes, dh)
    acc, cnt = _sc_scatter(table, src, tgt, n_nodes, n_spad, dh, steps,
                           tiles_per_type)
    return _combine(acc, cnt, n_nodes)
